# baseline (speedup 1.0000x reference)
import functools
import jax
import jax.numpy as jnp
from jax import lax
from jax.experimental import pallas as pl
from jax.experimental.pallas import tpu as pltpu
from jax.experimental.pallas import tpu_sc as plsc

SEQ = 200
DIM = 16
BATCH = 4096
NW = 32
SBLK = SEQ // 8
BBLK = BATCH // 128
NPAIR = SBLK * BBLK // NW
VOCAB = 1000000
PITCH = 129


def _make_transpose():
    mesh = plsc.VectorSubcoreMesh(core_axis_name="c", subcore_axis_name="s")
    NFULL = VOCAB // 128
    TAIL = VOCAB - NFULL * 128
    XTRA = NFULL - (NFULL // NW) * NW

    @functools.partial(
        pl.kernel,
        out_type=jax.ShapeDtypeStruct((VOCAB // 64, 8, 128), jnp.float32),
        mesh=mesh,
        compiler_params=pltpu.CompilerParams(
            use_tc_tiling_on_sc=True, needs_layout_passes=False
        ),
        scratch_types=[
            pltpu.VMEM((128, 128), jnp.float32),
            pltpu.VMEM((16, PITCH), jnp.float32),
            pltpu.VMEM((8, 2, 8, 128), jnp.float32),
            pltpu.SemaphoreType.DMA,
            pltpu.SemaphoreType.DMA,
        ],
    )
    def transpose_table(tabt_hbm, out_hbm, a_c, a_v, o_v, gsem, osem):
        wid = lax.axis_index("s") * 2 + lax.axis_index("c")
        nblk = jnp.where(wid < XTRA, NFULL // NW + 1, NFULL // NW)
        iota = jnp.arange(16, dtype=jnp.int32)

        def fire_in(r0, j):
            pltpu.async_copy(
                tabt_hbm.at[pl.ds(0, 16), pl.ds(j * 128, 128)],
                a_c.at[pl.ds(r0, 16)],
                gsem,
            )

        def drain_in(r0):
            pltpu.make_async_copy(
                tabt_hbm.at[pl.ds(0, 16), pl.ds(0, 128)],
                a_c.at[pl.ds(r0, 16)],
                gsem,
            ).wait()

        def wait_out():
            pltpu.make_async_copy(
                o_v.at[0], out_hbm.at[pl.ds(0, 2)], osem
            ).wait()

        for kk in range(8):
            fire_in(kk * 16, kk * NW + wid)

        def blk_body(k, carry):
            def work():
                r0 = (k & 7) * 16
                oslot = k & 7
                j = k * NW + wid
                drain_in(r0)
                pl.when(k >= 8)(wait_out)

                def s_body(r, c1):
                    for g in range(8):
                        a_v[r, pl.ds(g * 16, 16)] = a_c[r0 + r, pl.ds(g * 16, 16)]
                    return c1

                lax.fori_loop(0, 16, s_body, 0)

                def t_body(t0, c2):
                    tv = jnp.full((16,), 0, jnp.int32) + t0 * 8
                    p = t0 >> 3
                    a = t0 & 7
                    for r in range(8):
                        v = plsc.load_gather(a_v, [iota, tv + r])
                        o_v[oslot, p, a, pl.ds(r * 16, 16)] = v
                    return c2

                lax.fori_loop(0, 16, t_body, 0)
                pltpu.async_copy(o_v.at[oslot], out_hbm.at[pl.ds(2 * j, 2)], osem)

                @pl.when(k + 8 < nblk)
                def _():
                    fire_in(r0, (k + 8) * NW + wid)

            pl.when(k < nblk)(work)
            return carry

        lax.fori_loop(0, NFULL // NW + 1, blk_body, 0)
        for _ in range(8):
            wait_out()

        @pl.when(wid == XTRA)
        def _():
            for r in range(16):
                pltpu.sync_copy(
                    tabt_hbm.at[r, pl.ds(NFULL * 128, TAIL)],
                    a_v.at[r, pl.ds(0, TAIL)],
                )

            def t_body(t0, c2):
                tv = jnp.full((16,), 0, jnp.int32) + t0 * 8
                for r in range(8):
                    v = plsc.load_gather(a_v, [iota, tv + r])
                    o_v[0, 0, t0 & 7, pl.ds(r * 16, 16)] = v
                return c2

            lax.fori_loop(0, TAIL // 8, t_body, 0)
            pltpu.sync_copy(o_v.at[0, 0], out_hbm.at[2 * NFULL])

    return transpose_table


def _make_kernel():
    mesh = plsc.VectorSubcoreMesh(core_axis_name="c", subcore_axis_name="s")

    @functools.partial(
        pl.kernel,
        out_type=jax.ShapeDtypeStruct((SEQ, 2, BBLK, 8, 128), jnp.float32),
        mesh=mesh,
        compiler_params=pltpu.CompilerParams(
            use_tc_tiling_on_sc=False, needs_layout_passes=False
        ),
        scratch_types=[
            pltpu.VMEM((2, 1024), jnp.int32),
            pltpu.VMEM((2048, DIM), jnp.float32),
            pltpu.VMEM((2, 128, PITCH), jnp.float32),
            pltpu.VMEM((SEQ, DIM), jnp.float32),
            pltpu.SemaphoreType.DMA,
            pltpu.SemaphoreType.DMA,
            pltpu.SemaphoreType.DMA,
        ],
    )
    def tok_pos_embed(x_hbm, table_hbm, pos_hbm, out_hbm,
                      idx_v, rows_v, out_v, pos_v, gsem, osem, isem):
        wid = lax.axis_index("s") * 2 + lax.axis_index("c")
        pltpu.sync_copy(pos_hbm, pos_v)
        iota = jnp.arange(16, dtype=jnp.int32)

        def fire_idx(i):
            p = wid * NPAIR + i
            pltpu.async_copy(x_hbm.at[p >> 5, p & 31], idx_v.at[i & 1], isem)

        def wait_idx():
            pltpu.make_async_copy(
                x_hbm.at[0, 0], idx_v.at[0], isem
            ).wait()

        def fire_gathers(slot):
            for o in range(8):
                pltpu.async_copy(
                    table_hbm.at[idx_v.at[slot, pl.ds(o * 128, 128)]],
                    rows_v.at[pl.ds(slot * 1024 + o * 128, 128)],
                    gsem,
                )

        def drain_gathers(slot):
            for o in range(8):
                pltpu.make_async_copy(
                    table_hbm.at[idx_v.at[0, pl.ds(0, 128)]],
                    rows_v.at[pl.ds(slot * 1024 + o * 128, 128)],
                    gsem,
                ).wait()

        def out_dmas(fire, oslot, sblk, bblk):
            for ssub in range(8):
                for dblk in range(2):
                    src = out_v.at[oslot, pl.ds(ssub * 16 + dblk * 8, 8), pl.ds(0, 128)]
                    dst = out_hbm.at[sblk * 8 + ssub, dblk, bblk]
                    if fire:
                        pltpu.async_copy(src, dst, osem)
                    else:
                        pltpu.make_async_copy(src, dst, osem).wait()

        fire_idx(0)
        wait_idx()
        fire_gathers(0)
        fire_idx(1)

        def pair_body(i, carry):
            cur = i & 1
            p = wid * NPAIR + i
            sblk = p >> 5
            bblk = p & 31

            drain_gathers(cur)

            @pl.when(i + 2 < NPAIR)
            def _():
                fire_idx(i + 2)

            @pl.when(i + 1 < NPAIR)
            def _():
                wait_idx()
                fire_gathers(1 - cur)

            @pl.when(i >= 2)
            def _():
                out_dmas(False, cur, 0, 0)

            def ssub_body(ssub, c2):
                prow = pos_v[sblk * 8 + ssub]
                rowidx = iota + ssub * 16
                rbase = cur * 1024 + ssub * 128

                def q_body(q, c3):
                    qv = jnp.full((16,), 0, jnp.int32) + q * 8
                    for r in range(8):
                        v = rows_v[rbase + q * 8 + r] + prow
                        plsc.store_scatter(out_v.at[cur], [rowidx, qv + r], v)
                    return c3

                lax.fori_loop(0, 16, q_body, 0)
                return c2

            lax.fori_loop(0, 8, ssub_body, 0)
            out_dmas(True, cur, sblk, bblk)
            return carry

        lax.fori_loop(0, NPAIR, pair_body, 0)
        out_dmas(False, 0, 0, 0)
        out_dmas(False, 1, 0, 0)

    return tok_pos_embed


_kernel = _make_kernel()
_transpose = _make_transpose()


@jax.jit
def kernel(inputs, token_table, pos_table):
    x = (
        inputs.astype(jnp.int32)
        .reshape(BBLK, 128, SBLK, 8)
        .transpose(2, 0, 3, 1)
        .reshape(SBLK, BBLK, 1024)
    )
    tt = _transpose(token_table.T).reshape(VOCAB, DIM)
    out_phys = _kernel(x, tt, pos_table)
    return out_phys.transpose(2, 4, 0, 1, 3).reshape(BATCH, SEQ, DIM)


# speedup vs baseline: 1.1964x; 1.1964x over previous
import functools
import jax
import jax.numpy as jnp
from jax import lax
from jax.experimental import pallas as pl
from jax.experimental.pallas import tpu as pltpu
from jax.experimental.pallas import tpu_sc as plsc

SEQ = 200
DIM = 16
BATCH = 4096
NW = 32
SBLK = SEQ // 8
BBLK = BATCH // 128
NPAIR = SBLK * BBLK // NW
VOCAB = 1000000
PITCH = 136


def _make_transpose():
    mesh = plsc.VectorSubcoreMesh(core_axis_name="c", subcore_axis_name="s")
    NFULL = VOCAB // 128
    TAIL = VOCAB - NFULL * 128
    XTRA = NFULL - (NFULL // NW) * NW

    @functools.partial(
        pl.kernel,
        out_type=jax.ShapeDtypeStruct((VOCAB // 64, 8, 128), jnp.float32),
        mesh=mesh,
        compiler_params=pltpu.CompilerParams(
            use_tc_tiling_on_sc=True, needs_layout_passes=False
        ),
        scratch_types=[
            pltpu.VMEM((128, PITCH), jnp.float32),
            pltpu.VMEM((8, 2, 8, 128), jnp.float32),
            pltpu.SemaphoreType.DMA,
            pltpu.SemaphoreType.DMA,
        ],
    )
    def transpose_table(tabt_hbm, out_hbm, a_v, o_v, gsem, osem):
        wid = lax.axis_index("s") * 2 + lax.axis_index("c")
        nblk = jnp.where(wid < XTRA, NFULL // NW + 1, NFULL // NW)
        iota = jnp.arange(16, dtype=jnp.int32)

        def fire_in(r0, j):
            pltpu.async_copy(
                tabt_hbm.at[pl.ds(0, 16), pl.ds(j * 128, 128)],
                a_v.at[pl.ds(r0, 16), pl.ds(0, 128)],
                gsem,
            )

        def drain_in(r0):
            pltpu.make_async_copy(
                tabt_hbm.at[pl.ds(0, 16), pl.ds(0, 128)],
                a_v.at[pl.ds(r0, 16), pl.ds(0, 128)],
                gsem,
            ).wait()

        def wait_out():
            pltpu.make_async_copy(
                o_v.at[0], out_hbm.at[pl.ds(0, 2)], osem
            ).wait()

        for kk in range(8):
            fire_in(kk * 16, kk * NW + wid)

        def blk_body(k, carry):
            def work():
                r0 = (k & 7) * 16
                oslot = k & 7
                j = k * NW + wid
                drain_in(r0)
                pl.when(k >= 8)(wait_out)

                rowv = iota + r0

                def t_body(t0, c2):
                    tv = jnp.full((16,), 0, jnp.int32) + t0 * 8
                    p = t0 >> 3
                    a = t0 & 7
                    for r in range(8):
                        v = plsc.load_gather(a_v, [rowv, tv + r])
                        o_v[oslot, p, a, pl.ds(r * 16, 16)] = v
                    return c2

                lax.fori_loop(0, 16, t_body, 0)
                pltpu.async_copy(o_v.at[oslot], out_hbm.at[pl.ds(2 * j, 2)], osem)

                @pl.when(k + 8 < nblk)
                def _():
                    fire_in(r0, (k + 8) * NW + wid)

            pl.when(k < nblk)(work)
            return carry

        lax.fori_loop(0, NFULL // NW + 1, blk_body, 0)
        for _ in range(8):
            wait_out()

        @pl.when(wid == XTRA)
        def _():
            for r in range(16):
                pltpu.sync_copy(
                    tabt_hbm.at[r, pl.ds(NFULL * 128, TAIL)],
                    a_v.at[r, pl.ds(0, TAIL)],
                )

            def t_body(t0, c2):
                tv = jnp.full((16,), 0, jnp.int32) + t0 * 8
                for r in range(8):
                    v = plsc.load_gather(a_v, [iota, tv + r])
                    o_v[0, 0, t0 & 7, pl.ds(r * 16, 16)] = v
                return c2

            lax.fori_loop(0, TAIL // 8, t_body, 0)
            pltpu.sync_copy(o_v.at[0, 0], out_hbm.at[2 * NFULL])

    return transpose_table


def _make_kernel():
    mesh = plsc.VectorSubcoreMesh(core_axis_name="c", subcore_axis_name="s")

    @functools.partial(
        pl.kernel,
        out_type=jax.ShapeDtypeStruct((SEQ, 2, BBLK, 8, 128), jnp.float32),
        mesh=mesh,
        compiler_params=pltpu.CompilerParams(
            use_tc_tiling_on_sc=False, needs_layout_passes=False
        ),
        scratch_types=[
            pltpu.VMEM((2, 1024), jnp.int32),
            pltpu.VMEM((2048, DIM), jnp.float32),
            pltpu.VMEM((2, 128, PITCH), jnp.float32),
            pltpu.VMEM((SEQ, DIM), jnp.float32),
            pltpu.SemaphoreType.DMA,
            pltpu.SemaphoreType.DMA,
            pltpu.SemaphoreType.DMA,
        ],
    )
    def tok_pos_embed(x_hbm, table_hbm, pos_hbm, out_hbm,
                      idx_v, rows_v, out_v, pos_v, gsem, osem, isem):
        wid = lax.axis_index("s") * 2 + lax.axis_index("c")
        pltpu.sync_copy(pos_hbm, pos_v)
        iota = jnp.arange(16, dtype=jnp.int32)

        def fire_idx(i):
            p = wid * NPAIR + i
            pltpu.async_copy(x_hbm.at[p >> 5, p & 31], idx_v.at[i & 1], isem)

        def wait_idx():
            pltpu.make_async_copy(
                x_hbm.at[0, 0], idx_v.at[0], isem
            ).wait()

        def fire_gathers(slot):
            for o in range(8):
                pltpu.async_copy(
                    table_hbm.at[idx_v.at[slot, pl.ds(o * 128, 128)]],
                    rows_v.at[pl.ds(slot * 1024 + o * 128, 128)],
                    gsem,
                )

        def drain_gathers(slot):
            for o in range(8):
                pltpu.make_async_copy(
                    table_hbm.at[idx_v.at[0, pl.ds(0, 128)]],
                    rows_v.at[pl.ds(slot * 1024 + o * 128, 128)],
                    gsem,
                ).wait()

        def out_dmas(fire, oslot, sblk, bblk):
            for ssub in range(8):
                for dblk in range(2):
                    src = out_v.at[oslot, pl.ds(ssub * 16 + dblk * 8, 8), pl.ds(0, 128)]
                    dst = out_hbm.at[sblk * 8 + ssub, dblk, bblk]
                    if fire:
                        pltpu.async_copy(src, dst, osem)
                    else:
                        pltpu.make_async_copy(src, dst, osem).wait()

        fire_idx(0)
        wait_idx()
        fire_gathers(0)
        fire_idx(1)

        def pair_body(i, carry):
            cur = i & 1
            p = wid * NPAIR + i
            sblk = p >> 5
            bblk = p & 31

            drain_gathers(cur)

            @pl.when(i + 2 < NPAIR)
            def _():
                fire_idx(i + 2)

            @pl.when(i + 1 < NPAIR)
            def _():
                wait_idx()
                fire_gathers(1 - cur)

            @pl.when(i >= 2)
            def _():
                out_dmas(False, cur, 0, 0)

            def ssub_body(ssub, c2):
                prow = pos_v[sblk * 8 + ssub]
                rowidx = iota + ssub * 16
                rbase = cur * 1024 + ssub * 128

                def q_body(q, c3):
                    qv = jnp.full((16,), 0, jnp.int32) + q * 8
                    for r in range(8):
                        v = rows_v[rbase + q * 8 + r] + prow
                        plsc.store_scatter(out_v.at[cur], [rowidx, qv + r], v)
                    return c3

                lax.fori_loop(0, 16, q_body, 0)
                return c2

            lax.fori_loop(0, 8, ssub_body, 0)
            out_dmas(True, cur, sblk, bblk)
            return carry

        lax.fori_loop(0, NPAIR, pair_body, 0)
        out_dmas(False, 0, 0, 0)
        out_dmas(False, 1, 0, 0)

    return tok_pos_embed


_kernel = _make_kernel()
_transpose = _make_transpose()


@jax.jit
def kernel(inputs, token_table, pos_table):
    x = (
        inputs.astype(jnp.int32)
        .reshape(BBLK, 128, SBLK, 8)
        .transpose(2, 0, 3, 1)
        .reshape(SBLK, BBLK, 1024)
    )
    tt = _transpose(token_table.T).reshape(VOCAB, DIM)
    out_phys = _kernel(x, tt, pos_table)
    return out_phys.transpose(2, 4, 0, 1, 3).reshape(BATCH, SEQ, DIM)


# speedup vs baseline: 2.2057x; 1.8436x over previous
import functools
import jax
import jax.numpy as jnp
from jax import lax
from jax.experimental import pallas as pl
from jax.experimental.pallas import tpu as pltpu
from jax.experimental.pallas import tpu_sc as plsc

SEQ = 200
DIM = 16
BATCH = 4096
NW = 32
SBLK = SEQ // 8
BBLK = BATCH // 128
NPAIR = SBLK * BBLK // NW
VOCAB = 1000000
PITCH = 136


def _make_transpose():
    mesh = plsc.VectorSubcoreMesh(core_axis_name="c", subcore_axis_name="s")
    NFULL = VOCAB // 128
    TAIL = VOCAB - NFULL * 128
    XTRA = NFULL - (NFULL // NW) * NW

    @functools.partial(
        pl.kernel,
        out_type=jax.ShapeDtypeStruct((VOCAB // 64, 8, 128), jnp.float32),
        mesh=mesh,
        compiler_params=pltpu.CompilerParams(
            use_tc_tiling_on_sc=True, needs_layout_passes=False
        ),
        scratch_types=[
            pltpu.VMEM((128, PITCH), jnp.float32),
            pltpu.VMEM((8, 2, 8, 128), jnp.float32),
            pltpu.SemaphoreType.DMA,
            pltpu.SemaphoreType.DMA,
        ],
    )
    def transpose_table(tabt_hbm, out_hbm, a_v, o_v, gsem, osem):
        wid = lax.axis_index("s") * 2 + lax.axis_index("c")
        nblk = jnp.where(wid < XTRA, NFULL // NW + 1, NFULL // NW)
        iota = jnp.arange(16, dtype=jnp.int32)

        def fire_in(r0, j):
            pltpu.async_copy(
                tabt_hbm.at[pl.ds(0, 16), pl.ds(j * 128, 128)],
                a_v.at[pl.ds(r0, 16), pl.ds(0, 128)],
                gsem,
            )

        def drain_in(r0):
            pltpu.make_async_copy(
                tabt_hbm.at[pl.ds(0, 16), pl.ds(0, 128)],
                a_v.at[pl.ds(r0, 16), pl.ds(0, 128)],
                gsem,
            ).wait()

        def wait_out():
            pltpu.make_async_copy(
                o_v.at[0], out_hbm.at[pl.ds(0, 2)], osem
            ).wait()

        for kk in range(8):
            fire_in(kk * 16, kk * NW + wid)

        def blk_body(k, carry):
            def work():
                r0 = (k & 7) * 16
                oslot = k & 7
                j = k * NW + wid
                drain_in(r0)
                pl.when(k >= 8)(wait_out)

                rowv = iota + r0

                @plsc.parallel_loop(0, 16, unroll=4)
                def t_body(t0):
                    tv = jnp.full((16,), 0, jnp.int32) + t0 * 8
                    p = t0 >> 3
                    a = t0 & 7
                    for r in range(8):
                        v = plsc.load_gather(a_v, [rowv, tv + r])
                        o_v[oslot, p, a, pl.ds(r * 16, 16)] = v
                pltpu.async_copy(o_v.at[oslot], out_hbm.at[pl.ds(2 * j, 2)], osem)

                @pl.when(k + 8 < nblk)
                def _():
                    fire_in(r0, (k + 8) * NW + wid)

            pl.when(k < nblk)(work)
            return carry

        lax.fori_loop(0, NFULL // NW + 1, blk_body, 0)
        for _ in range(8):
            wait_out()

        @pl.when(wid == XTRA)
        def _():
            for r in range(16):
                pltpu.sync_copy(
                    tabt_hbm.at[r, pl.ds(NFULL * 128, TAIL)],
                    a_v.at[r, pl.ds(0, TAIL)],
                )

            def t_body(t0, c2):
                tv = jnp.full((16,), 0, jnp.int32) + t0 * 8
                for r in range(8):
                    v = plsc.load_gather(a_v, [iota, tv + r])
                    o_v[0, 0, t0 & 7, pl.ds(r * 16, 16)] = v
                return c2

            lax.fori_loop(0, TAIL // 8, t_body, 0)
            pltpu.sync_copy(o_v.at[0, 0], out_hbm.at[2 * NFULL])

    return transpose_table


def _make_kernel():
    mesh = plsc.VectorSubcoreMesh(core_axis_name="c", subcore_axis_name="s")

    @functools.partial(
        pl.kernel,
        out_type=jax.ShapeDtypeStruct((SEQ, 2, BBLK, 8, 128), jnp.float32),
        mesh=mesh,
        compiler_params=pltpu.CompilerParams(
            use_tc_tiling_on_sc=False, needs_layout_passes=False
        ),
        scratch_types=[
            pltpu.VMEM((2, 1024), jnp.int32),
            pltpu.VMEM((2048, DIM), jnp.float32),
            pltpu.VMEM((2, 128, PITCH), jnp.float32),
            pltpu.VMEM((SEQ, DIM), jnp.float32),
            pltpu.SemaphoreType.DMA,
            pltpu.SemaphoreType.DMA,
            pltpu.SemaphoreType.DMA,
        ],
    )
    def tok_pos_embed(x_hbm, table_hbm, pos_hbm, out_hbm,
                      idx_v, rows_v, out_v, pos_v, gsem, osem, isem):
        wid = lax.axis_index("s") * 2 + lax.axis_index("c")
        pltpu.sync_copy(pos_hbm, pos_v)
        iota = jnp.arange(16, dtype=jnp.int32)

        def fire_idx(i):
            p = wid * NPAIR + i
            pltpu.async_copy(x_hbm.at[p >> 5, p & 31], idx_v.at[i & 1], isem)

        def wait_idx():
            pltpu.make_async_copy(
                x_hbm.at[0, 0], idx_v.at[0], isem
            ).wait()

        def fire_gathers(slot):
            for o in range(8):
                pltpu.async_copy(
                    table_hbm.at[idx_v.at[slot, pl.ds(o * 128, 128)]],
                    rows_v.at[pl.ds(slot * 1024 + o * 128, 128)],
                    gsem,
                )

        def drain_gathers(slot):
            for o in range(8):
                pltpu.make_async_copy(
                    table_hbm.at[idx_v.at[0, pl.ds(0, 128)]],
                    rows_v.at[pl.ds(slot * 1024 + o * 128, 128)],
                    gsem,
                ).wait()

        def out_dmas(fire, oslot, sblk, bblk):
            for ssub in range(8):
                for dblk in range(2):
                    src = out_v.at[oslot, pl.ds(ssub * 16 + dblk * 8, 8), pl.ds(0, 128)]
                    dst = out_hbm.at[sblk * 8 + ssub, dblk, bblk]
                    if fire:
                        pltpu.async_copy(src, dst, osem)
                    else:
                        pltpu.make_async_copy(src, dst, osem).wait()

        fire_idx(0)
        wait_idx()
        fire_gathers(0)
        fire_idx(1)

        def pair_body(i, carry):
            cur = i & 1
            p = wid * NPAIR + i
            sblk = p >> 5
            bblk = p & 31

            drain_gathers(cur)

            @pl.when(i + 2 < NPAIR)
            def _():
                fire_idx(i + 2)

            @pl.when(i + 1 < NPAIR)
            def _():
                wait_idx()
                fire_gathers(1 - cur)

            @pl.when(i >= 2)
            def _():
                out_dmas(False, cur, 0, 0)

            def ssub_body(ssub, c2):
                prow = pos_v[sblk * 8 + ssub]
                rowidx = iota + ssub * 16
                rbase = cur * 1024 + ssub * 128

                @plsc.parallel_loop(0, 16, unroll=4)
                def q_body(q):
                    qv = jnp.full((16,), 0, jnp.int32) + q * 8
                    for r in range(8):
                        v = rows_v[rbase + q * 8 + r] + prow
                        plsc.store_scatter(out_v.at[cur], [rowidx, qv + r], v)

                return c2

            lax.fori_loop(0, 8, ssub_body, 0)
            out_dmas(True, cur, sblk, bblk)
            return carry

        lax.fori_loop(0, NPAIR, pair_body, 0)
        out_dmas(False, 0, 0, 0)
        out_dmas(False, 1, 0, 0)

    return tok_pos_embed


_kernel = _make_kernel()
_transpose = _make_transpose()


@jax.jit
def kernel(inputs, token_table, pos_table):
    x = (
        inputs.astype(jnp.int32)
        .reshape(BBLK, 128, SBLK, 8)
        .transpose(2, 0, 3, 1)
        .reshape(SBLK, BBLK, 1024)
    )
    tt = _transpose(token_table.T).reshape(VOCAB, DIM)
    out_phys = _kernel(x, tt, pos_table)
    return out_phys.transpose(2, 4, 0, 1, 3).reshape(BATCH, SEQ, DIM)


# speedup vs baseline: 2.2097x; 1.0019x over previous
import functools
import jax
import jax.numpy as jnp
from jax import lax
from jax.experimental import pallas as pl
from jax.experimental.pallas import tpu as pltpu
from jax.experimental.pallas import tpu_sc as plsc

SEQ = 200
DIM = 16
BATCH = 4096
NW = 32
SBLK = SEQ // 8
BBLK = BATCH // 128
NPAIR = SBLK * BBLK // NW
VOCAB = 1000000
PITCH = 136


def _make_transpose():
    mesh = plsc.VectorSubcoreMesh(core_axis_name="c", subcore_axis_name="s")
    NFULL = VOCAB // 128
    TAIL = VOCAB - NFULL * 128
    XTRA = NFULL - (NFULL // NW) * NW

    @functools.partial(
        pl.kernel,
        out_type=jax.ShapeDtypeStruct((VOCAB // 64, 8, 128), jnp.float32),
        mesh=mesh,
        compiler_params=pltpu.CompilerParams(
            use_tc_tiling_on_sc=True, needs_layout_passes=False
        ),
        scratch_types=[
            pltpu.VMEM((128, 128), jnp.float32),
            pltpu.VMEM((8, 2, 8, 128), jnp.float32),
            pltpu.SemaphoreType.DMA,
            pltpu.SemaphoreType.DMA,
        ],
    )
    def transpose_table(tabt_hbm, out_hbm, a_v, o_v, gsem, osem):
        wid = lax.axis_index("s") * 2 + lax.axis_index("c")
        nblk = jnp.where(wid < XTRA, NFULL // NW + 1, NFULL // NW)
        iota = jnp.arange(16, dtype=jnp.int32)

        def fire_in(r0, j):
            pltpu.async_copy(
                tabt_hbm.at[pl.ds(0, 16), pl.ds(j * 128, 128)],
                a_v.at[pl.ds(r0, 16)],
                gsem,
            )

        def drain_in(r0):
            pltpu.make_async_copy(
                tabt_hbm.at[pl.ds(0, 16), pl.ds(0, 128)],
                a_v.at[pl.ds(r0, 16)],
                gsem,
            ).wait()

        def wait_out():
            pltpu.make_async_copy(
                o_v.at[0], out_hbm.at[pl.ds(0, 2)], osem
            ).wait()

        for kk in range(8):
            fire_in(kk * 16, kk * NW + wid)

        def blk_body(k, carry):
            def work():
                r0 = (k & 7) * 16
                oslot = k & 7
                j = k * NW + wid
                drain_in(r0)
                pl.when(k >= 8)(wait_out)

                rowv = iota + r0

                @plsc.parallel_loop(0, 16, unroll=4)
                def t_body(t0):
                    tv = jnp.full((16,), 0, jnp.int32) + t0 * 8
                    p = t0 >> 3
                    a = t0 & 7
                    for r in range(8):
                        v = plsc.load_gather(a_v, [rowv, tv + r])
                        o_v[oslot, p, a, pl.ds(r * 16, 16)] = v
                pltpu.async_copy(o_v.at[oslot], out_hbm.at[pl.ds(2 * j, 2)], osem)

                @pl.when(k + 8 < nblk)
                def _():
                    fire_in(r0, (k + 8) * NW + wid)

            pl.when(k < nblk)(work)
            return carry

        lax.fori_loop(0, NFULL // NW + 1, blk_body, 0)
        for _ in range(8):
            wait_out()

        @pl.when(wid == XTRA)
        def _():
            for r in range(16):
                pltpu.sync_copy(
                    tabt_hbm.at[r, pl.ds(NFULL * 128, TAIL)],
                    a_v.at[r, pl.ds(0, TAIL)],
                )

            def t_body(t0, c2):
                tv = jnp.full((16,), 0, jnp.int32) + t0 * 8
                for r in range(8):
                    v = plsc.load_gather(a_v, [iota, tv + r])
                    o_v[0, 0, t0 & 7, pl.ds(r * 16, 16)] = v
                return c2

            lax.fori_loop(0, TAIL // 8, t_body, 0)
            pltpu.sync_copy(o_v.at[0, 0], out_hbm.at[2 * NFULL])

    return transpose_table


def _make_kernel():
    mesh = plsc.VectorSubcoreMesh(core_axis_name="c", subcore_axis_name="s")

    @functools.partial(
        pl.kernel,
        out_type=jax.ShapeDtypeStruct((SEQ, 2, BBLK, 8, 128), jnp.float32),
        mesh=mesh,
        compiler_params=pltpu.CompilerParams(
            use_tc_tiling_on_sc=False, needs_layout_passes=False
        ),
        scratch_types=[
            pltpu.VMEM((2, 1024), jnp.int32),
            pltpu.VMEM((2048, DIM), jnp.float32),
            pltpu.VMEM((2, 128, PITCH), jnp.float32),
            pltpu.VMEM((SEQ, DIM), jnp.float32),
            pltpu.SemaphoreType.DMA,
            pltpu.SemaphoreType.DMA,
            pltpu.SemaphoreType.DMA,
        ],
    )
    def tok_pos_embed(x_hbm, table_hbm, pos_hbm, out_hbm,
                      idx_v, rows_v, out_v, pos_v, gsem, osem, isem):
        wid = lax.axis_index("s") * 2 + lax.axis_index("c")
        pltpu.sync_copy(pos_hbm, pos_v)
        iota = jnp.arange(16, dtype=jnp.int32)

        def fire_idx(i):
            p = wid * NPAIR + i
            pltpu.async_copy(x_hbm.at[p >> 5, p & 31], idx_v.at[i & 1], isem)

        def wait_idx():
            pltpu.make_async_copy(
                x_hbm.at[0, 0], idx_v.at[0], isem
            ).wait()

        def fire_gathers(slot):
            for o in range(8):
                pltpu.async_copy(
                    table_hbm.at[idx_v.at[slot, pl.ds(o * 128, 128)]],
                    rows_v.at[pl.ds(slot * 1024 + o * 128, 128)],
                    gsem,
                )

        def drain_gathers(slot):
            for o in range(8):
                pltpu.make_async_copy(
                    table_hbm.at[idx_v.at[0, pl.ds(0, 128)]],
                    rows_v.at[pl.ds(slot * 1024 + o * 128, 128)],
                    gsem,
                ).wait()

        def out_dmas(fire, oslot, sblk, bblk):
            for ssub in range(8):
                for dblk in range(2):
                    src = out_v.at[oslot, pl.ds(ssub * 16 + dblk * 8, 8), pl.ds(0, 128)]
                    dst = out_hbm.at[sblk * 8 + ssub, dblk, bblk]
                    if fire:
                        pltpu.async_copy(src, dst, osem)
                    else:
                        pltpu.make_async_copy(src, dst, osem).wait()

        fire_idx(0)
        wait_idx()
        fire_gathers(0)
        fire_idx(1)

        def pair_body(i, carry):
            cur = i & 1
            p = wid * NPAIR + i
            sblk = p >> 5
            bblk = p & 31

            drain_gathers(cur)

            @pl.when(i + 2 < NPAIR)
            def _():
                fire_idx(i + 2)

            @pl.when(i + 1 < NPAIR)
            def _():
                wait_idx()
                fire_gathers(1 - cur)

            @pl.when(i >= 2)
            def _():
                out_dmas(False, cur, 0, 0)

            def ssub_body(ssub, c2):
                prow = pos_v[sblk * 8 + ssub]
                rowidx = iota + ssub * 16
                rbase = cur * 1024 + ssub * 128

                @plsc.parallel_loop(0, 16, unroll=4)
                def q_body(q):
                    qv = jnp.full((16,), 0, jnp.int32) + q * 8
                    for r in range(8):
                        v = rows_v[rbase + q * 8 + r] + prow
                        plsc.store_scatter(out_v.at[cur], [rowidx, qv + r], v)

                return c2

            lax.fori_loop(0, 8, ssub_body, 0)
            out_dmas(True, cur, sblk, bblk)
            return carry

        lax.fori_loop(0, NPAIR, pair_body, 0)
        out_dmas(False, 0, 0, 0)
        out_dmas(False, 1, 0, 0)

    return tok_pos_embed


_kernel = _make_kernel()
_transpose = _make_transpose()


@jax.jit
def kernel(inputs, token_table, pos_table):
    x = (
        inputs.astype(jnp.int32)
        .reshape(BBLK, 128, SBLK, 8)
        .transpose(2, 0, 3, 1)
        .reshape(SBLK, BBLK, 1024)
    )
    tt = _transpose(token_table.T).reshape(VOCAB, DIM)
    out_phys = _kernel(x, tt, pos_table)
    return out_phys.transpose(2, 4, 0, 1, 3).reshape(BATCH, SEQ, DIM)
